# Initial kernel scaffold; baseline (speedup 1.0000x reference)
import jax
import jax.numpy as jnp
from jax.experimental import pallas as pl


def kernel(char_code, num_bytes, is_letter, is_number, is_punctuation, is_whitespace, c_table, n_table, l_table, num_table, p_table, w_table, dense_kernel, dense_bias):
    raise NotImplementedError("write your pallas kernel here")


# speedup vs baseline: 7.4648x; 7.4648x over previous
import functools

import jax
import jax.numpy as jnp
from jax import lax
from jax.experimental import pallas as pl
from jax.experimental.pallas import tpu as pltpu
from jax.experimental.pallas import tpu_sc as plsc

B, P, H, W = 16, 4, 64, 128
N = B * P * H * W
D = 256
NC, NS = 2, 16
NW = NC * NS
NT = N // NW
C = 128
G = NT // C


def _prep_body(c_ref, n_ref, l_ref, num_ref, p_ref, w_ref, dk_ref, b_ref,
               t1_ref, t2_ref):
    dk = dk_ref[...]
    t1_ref[...] = jnp.dot(c_ref[...], dk[0:64, :],
                          preferred_element_type=jnp.float32)
    n_proj = jnp.dot(n_ref[...], dk[64:80, :],
                     preferred_element_type=jnp.float32)
    l_proj = jnp.dot(l_ref[...], dk[80:112, :],
                     preferred_element_type=jnp.float32)
    num_proj = jnp.dot(num_ref[...], dk[112:144, :],
                       preferred_element_type=jnp.float32)
    p_proj = jnp.dot(p_ref[...], dk[144:176, :],
                     preferred_element_type=jnp.float32)
    w_proj = jnp.dot(w_ref[...], dk[176:208, :],
                     preferred_element_type=jnp.float32)

    idx = lax.broadcasted_iota(jnp.int32, (80, 1), 0)
    nb = idx // 16
    lbit = (idx // 8) % 2
    nbit = (idx // 4) % 2
    pbit = (idx // 2) % 2
    wbit = idx % 2

    acc = b_ref[...]
    for k in range(5):
        acc = acc + jnp.where(nb == k, 1.0, 0.0) * n_proj[k:k + 1, :]
    acc = acc + jnp.where(lbit == 1, l_proj[1:2, :], l_proj[0:1, :])
    acc = acc + jnp.where(nbit == 1, num_proj[1:2, :], num_proj[0:1, :])
    acc = acc + jnp.where(pbit == 1, p_proj[1:2, :], p_proj[0:1, :])
    acc = acc + jnp.where(wbit == 1, w_proj[1:2, :], w_proj[0:1, :])
    t2_ref[...] = acc


def _prep_tables(c_table, n_table, l_table, num_table, p_table, w_table,
                 dense_kernel, dense_bias):
    return pl.pallas_call(
        _prep_body,
        out_shape=[
            jax.ShapeDtypeStruct((300, D), jnp.float32),
            jax.ShapeDtypeStruct((80, D), jnp.float32),
        ],
    )(c_table, n_table, l_table, num_table, p_table, w_table,
      dense_kernel, dense_bias.reshape(1, D))


def _sc_body(cc, nb, il, inum, ip, iw, t1, t2, out,
             cc_v, nb_v, il_v, in_v, ip_v, iw_v, i1_v, i2_v,
             rows1, rows2, sem1, sem2):
    wid = lax.axis_index("s") * NC + lax.axis_index("c")
    base0 = wid * NT

    def step(g, carry):
        base = base0 + g * C
        pltpu.sync_copy(cc.at[pl.ds(base, C)], cc_v)
        pltpu.sync_copy(nb.at[pl.ds(base, C)], nb_v)
        pltpu.sync_copy(il.at[pl.ds(base, C)], il_v)
        pltpu.sync_copy(inum.at[pl.ds(base, C)], in_v)
        pltpu.sync_copy(ip.at[pl.ds(base, C)], ip_v)
        pltpu.sync_copy(iw.at[pl.ds(base, C)], iw_v)

        def ixbody(j, carry2):
            s = j * 16
            sl = pl.ds(s, 16)
            i1_v[sl] = lax.rem(cc_v[sl], 300)
            comb = nb_v[sl] * 16 + il_v[sl] * 8 + in_v[sl] * 4 \
                + ip_v[sl] * 2 + iw_v[sl]
            i2_v[sl] = comb
            return carry2

        lax.fori_loop(0, C // 16, ixbody, 0)

        cp1 = pltpu.async_copy(t1.at[i1_v], rows1, sem1)
        cp2 = pltpu.async_copy(t2.at[i2_v], rows2, sem2)
        cp1.wait()
        cp2.wait()

        def addbody(t, carry2):
            for u in range(D // 16):
                sl = pl.ds(u * 16, 16)
                rows1[t, sl] = rows1[t, sl] + rows2[t, sl]
            return carry2

        lax.fori_loop(0, C, addbody, 0)

        pltpu.sync_copy(rows1, out.at[pl.ds(base, C)])
        return carry

    lax.fori_loop(0, G, step, 0)


_sc_kernel = functools.partial(
    pl.kernel,
    mesh=plsc.VectorSubcoreMesh(core_axis_name="c", subcore_axis_name="s"),
    out_type=jax.ShapeDtypeStruct((N, D), jnp.float32),
    scratch_types=[
        pltpu.VMEM((C,), jnp.int32),
        pltpu.VMEM((C,), jnp.int32),
        pltpu.VMEM((C,), jnp.int32),
        pltpu.VMEM((C,), jnp.int32),
        pltpu.VMEM((C,), jnp.int32),
        pltpu.VMEM((C,), jnp.int32),
        pltpu.VMEM((C,), jnp.int32),
        pltpu.VMEM((C,), jnp.int32),
        pltpu.VMEM((C, D), jnp.float32),
        pltpu.VMEM((C, D), jnp.float32),
        pltpu.SemaphoreType.DMA,
        pltpu.SemaphoreType.DMA,
    ],
)(_sc_body)


def kernel(char_code, num_bytes, is_letter, is_number, is_punctuation,
           is_whitespace, c_table, n_table, l_table, num_table, p_table,
           w_table, dense_kernel, dense_bias):
    t1, t2 = _prep_tables(c_table, n_table, l_table, num_table, p_table,
                          w_table, dense_kernel, dense_bias)
    cc = char_code.reshape(N).astype(jnp.int32)
    nb = num_bytes.reshape(N).astype(jnp.int32)
    il = is_letter.reshape(N).astype(jnp.int32)
    inum = is_number.reshape(N).astype(jnp.int32)
    ip = is_punctuation.reshape(N).astype(jnp.int32)
    iw = is_whitespace.reshape(N).astype(jnp.int32)
    out = _sc_kernel(cc, nb, il, inum, ip, iw, t1, t2)
    return out.reshape(B, P, H, W, D)


# speedup vs baseline: 7.8508x; 1.0517x over previous
import functools

import jax
import jax.numpy as jnp
from jax import lax
from jax.experimental import pallas as pl
from jax.experimental.pallas import tpu as pltpu
from jax.experimental.pallas import tpu_sc as plsc

B, P, H, W = 16, 4, 64, 128
N = B * P * H * W
D = 256
NC, NS = 2, 16
NW = NC * NS
NT = N // NW
C = 64
IB = 1024
CB = IB // C
HB = CB // 2
NBLK = NT // IB


def _prep_body(c_ref, n_ref, l_ref, num_ref, p_ref, w_ref, dk_ref, b_ref,
               t1_ref, t2_ref):
    dk = dk_ref[...]
    t1_ref[...] = jnp.dot(c_ref[...], dk[0:64, :],
                          preferred_element_type=jnp.float32)
    n_proj = jnp.dot(n_ref[...], dk[64:80, :],
                     preferred_element_type=jnp.float32)
    l_proj = jnp.dot(l_ref[...], dk[80:112, :],
                     preferred_element_type=jnp.float32)
    num_proj = jnp.dot(num_ref[...], dk[112:144, :],
                       preferred_element_type=jnp.float32)
    p_proj = jnp.dot(p_ref[...], dk[144:176, :],
                     preferred_element_type=jnp.float32)
    w_proj = jnp.dot(w_ref[...], dk[176:208, :],
                     preferred_element_type=jnp.float32)

    idx = lax.broadcasted_iota(jnp.int32, (80, 1), 0)
    nb = idx // 16
    lbit = (idx // 8) % 2
    nbit = (idx // 4) % 2
    pbit = (idx // 2) % 2
    wbit = idx % 2

    acc = b_ref[...]
    for k in range(5):
        acc = acc + jnp.where(nb == k, 1.0, 0.0) * n_proj[k:k + 1, :]
    acc = acc + jnp.where(lbit == 1, l_proj[1:2, :], l_proj[0:1, :])
    acc = acc + jnp.where(nbit == 1, num_proj[1:2, :], num_proj[0:1, :])
    acc = acc + jnp.where(pbit == 1, p_proj[1:2, :], p_proj[0:1, :])
    acc = acc + jnp.where(wbit == 1, w_proj[1:2, :], w_proj[0:1, :])
    t2_ref[...] = acc


def _prep_tables(c_table, n_table, l_table, num_table, p_table, w_table,
                 dense_kernel, dense_bias):
    return pl.pallas_call(
        _prep_body,
        out_shape=[
            jax.ShapeDtypeStruct((300, D), jnp.float32),
            jax.ShapeDtypeStruct((80, D), jnp.float32),
        ],
    )(c_table, n_table, l_table, num_table, p_table, w_table,
      dense_kernel, dense_bias.reshape(1, D))


def _sc_body(cc, nb, il, inum, ip, iw, t1, t2, out,
             cc_v, nb_v, il_v, in_v, ip_v, iw_v, i1_b, i2_b,
             rA1, rA2, rB1, rB2, oS0, oS1,
             semIdx, sA1, sA2, sB1, sB2, sO0, sO1):
    wid = lax.axis_index("s") * NC + lax.axis_index("c")
    base0 = wid * NT
    rows = ((rA1, rA2, sA1, sA2), (rB1, rB2, sB1, sB2))
    ost = ((oS0, sO0), (oS1, sO1))

    def issue_gather(cl, set_i):
        r1, r2, s1, s2 = rows[set_i]
        off = cl * C
        pltpu.async_copy(t1.at[i1_b.at[pl.ds(off, C)]], r1, s1)
        pltpu.async_copy(t2.at[i2_b.at[pl.ds(off, C)]], r2, s2)

    def wait_gather(set_i):
        r1, r2, s1, s2 = rows[set_i]
        pltpu.make_async_copy(t1.at[i1_b.at[pl.ds(0, C)]], r1, s1).wait()
        pltpu.make_async_copy(t2.at[i2_b.at[pl.ds(0, C)]], r2, s2).wait()

    def add_and_store(cl, blk, set_i):
        r1, r2, _, _ = rows[set_i]
        ob, osem = ost[set_i]
        gc = blk * CB + cl

        @pl.when(gc >= 2)
        def _():
            pltpu.make_async_copy(ob, out.at[pl.ds(0, C)], osem).wait()

        def addbody(t, carry2):
            for u in range(D // 16):
                sl = pl.ds(u * 16, 16)
                ob[t, sl] = r1[t, sl] + r2[t, sl]
            return carry2

        lax.fori_loop(0, C, addbody, 0)
        base = base0 + blk * IB + cl * C
        pltpu.async_copy(ob, out.at[pl.ds(base, C)], osem)

    def block(blk, carry):
        bbase = base0 + blk * IB
        cps = [pltpu.async_copy(src.at[pl.ds(bbase, IB)], dst, semIdx)
               for src, dst in zip((cc, nb, il, inum, ip, iw),
                                   (cc_v, nb_v, il_v, in_v, ip_v, iw_v))]
        for cp in cps:
            cp.wait()

        def ixbody(j, carry2):
            sl = pl.ds(j * 16, 16)
            i1_b[sl] = lax.rem(cc_v[sl], 300)
            i2_b[sl] = nb_v[sl] * 16 + il_v[sl] * 8 + in_v[sl] * 4 \
                + ip_v[sl] * 2 + iw_v[sl]
            return carry2

        lax.fori_loop(0, IB // 16, ixbody, 0)
        issue_gather(0, 0)

        def pair(h, carry2):
            c0 = 2 * h
            issue_gather(c0 + 1, 1)
            wait_gather(0)
            add_and_store(c0, blk, 0)

            @pl.when(h < HB - 1)
            def _():
                issue_gather(c0 + 2, 0)

            wait_gather(1)
            add_and_store(c0 + 1, blk, 1)
            return carry2

        lax.fori_loop(0, HB, pair, 0)
        return carry

    lax.fori_loop(0, NBLK, block, 0)
    pltpu.make_async_copy(oS0, out.at[pl.ds(0, C)], sO0).wait()
    pltpu.make_async_copy(oS1, out.at[pl.ds(0, C)], sO1).wait()


_sc_kernel = functools.partial(
    pl.kernel,
    mesh=plsc.VectorSubcoreMesh(core_axis_name="c", subcore_axis_name="s"),
    out_type=jax.ShapeDtypeStruct((N, D), jnp.float32),
    scratch_types=[
        pltpu.VMEM((IB,), jnp.int32),
        pltpu.VMEM((IB,), jnp.int32),
        pltpu.VMEM((IB,), jnp.int32),
        pltpu.VMEM((IB,), jnp.int32),
        pltpu.VMEM((IB,), jnp.int32),
        pltpu.VMEM((IB,), jnp.int32),
        pltpu.VMEM((IB,), jnp.int32),
        pltpu.VMEM((IB,), jnp.int32),
        pltpu.VMEM((C, D), jnp.float32),
        pltpu.VMEM((C, D), jnp.float32),
        pltpu.VMEM((C, D), jnp.float32),
        pltpu.VMEM((C, D), jnp.float32),
        pltpu.VMEM((C, D), jnp.float32),
        pltpu.VMEM((C, D), jnp.float32),
        pltpu.SemaphoreType.DMA,
        pltpu.SemaphoreType.DMA,
        pltpu.SemaphoreType.DMA,
        pltpu.SemaphoreType.DMA,
        pltpu.SemaphoreType.DMA,
        pltpu.SemaphoreType.DMA,
        pltpu.SemaphoreType.DMA,
    ],
)(_sc_body)


def kernel(char_code, num_bytes, is_letter, is_number, is_punctuation,
           is_whitespace, c_table, n_table, l_table, num_table, p_table,
           w_table, dense_kernel, dense_bias):
    t1, t2 = _prep_tables(c_table, n_table, l_table, num_table, p_table,
                          w_table, dense_kernel, dense_bias)
    cc = char_code.reshape(N).astype(jnp.int32)
    nb = num_bytes.reshape(N).astype(jnp.int32)
    il = is_letter.reshape(N).astype(jnp.int32)
    inum = is_number.reshape(N).astype(jnp.int32)
    ip = is_punctuation.reshape(N).astype(jnp.int32)
    iw = is_whitespace.reshape(N).astype(jnp.int32)
    out = _sc_kernel(cc, nb, il, inum, ip, iw, t1, t2)
    return out.reshape(B, P, H, W, D)
